# baseline (speedup 1.0000x reference)
import numpy as np
import jax
import jax.numpy as jnp
from jax.experimental import pallas as pl
from jax.experimental.pallas import tpu as pltpu

_F32 = jnp.float32


def _shift_matrix(win, pout):
    e = np.zeros((3, win, 2, pout), np.float32)
    for dx in range(3):
        for q in range(2):
            for p in range(pout):
                e[dx, 2 * p + q + dx, q, p] = 1.0
    return e


def _conv_toeplitz(w_mat, cin, win, pout, cin_major, pad_lanes):
    cout = w_mat.shape[1]
    taps = w_mat.reshape(3, 3, cin, cout)
    e = jnp.asarray(_shift_matrix(win, pout))
    blocks = []
    for dy in range(3):
        t = taps[dy]
        if cin_major:
            wd = jnp.einsum("dxqp,dco->cxqpo", e, t)
            wd = wd.reshape(cin * win, 2, pout * cout)
        else:
            wd = jnp.einsum("dxqp,dco->xcqpo", e, t)
            wd = wd.reshape(win * cin, 2, pout * cout)
        if pad_lanes > pout * cout:
            wd = jnp.pad(wd, ((0, 0), (0, 0), (0, pad_lanes - pout * cout)))
        blocks.append(wd.reshape(wd.shape[0], -1))
    return jnp.concatenate(blocks, axis=1)


def _fused_net_kernel(x_ref, w1_ref, b1_ref, w2_ref, b2_ref, w3_ref, b3_ref,
                      wf1_ref, bf1_ref, wf2_ref, bf2_ref, wf3_ref, bf3_ref,
                      o_ref):
    tb = x_ref.shape[0]
    bf16 = jnp.bfloat16

    x = x_ref[...].reshape(tb * 32, 96)
    z = jnp.dot(x, w1_ref[...], preferred_element_type=_F32)
    z = z.reshape(tb, 32, 768)

    def s1(t, b, n):
        g, ko = t % 8, t // 8
        return z[:, g * 4 + ko: g * 4 + ko + n, 256 * b:256 * (b + 1)]

    p1 = []
    for h in range(4):
        n = 3 if h == 3 else 4
        c0 = s1(2 * h, 0, n) + s1(2 * h + 1, 1, n) + s1(2 * h + 2, 2, n)
        c1 = s1(2 * h + 1, 0, n) + s1(2 * h + 2, 1, n) + s1(2 * h + 3, 2, n)
        p = jnp.maximum(c0, c1)
        p = jnp.maximum(p[:, :, 0:128], p[:, :, 128:256])
        p1.append(jnp.maximum(p + b1_ref[0], 0.0).astype(bf16))

    z1 = jnp.zeros((tb, 1, 128), bf16)
    z3 = jnp.zeros((tb, 3, 128), bf16)
    z4 = jnp.zeros((tb, 4, 128), bf16)
    x2 = jnp.concatenate(
        [z1, p1[2], z3,
         z1, p1[3], z4,
         p1[0], z4,
         p1[1], z4], axis=1)
    zc = jnp.zeros((tb, 32, 16), bf16)
    x2 = jnp.concatenate([zc, x2, zc[:, :, 0:8]], axis=2)
    z = jnp.dot(x2.reshape(tb * 32, 152), w2_ref[...],
                preferred_element_type=_F32).reshape(tb, 32, 768)

    def s2(t, b):
        g, ko = t % 4, t // 4
        return z[:, g * 8 + ko: g * 8 + ko + 4, 256 * b:256 * (b + 1)]

    p2 = []
    for h in range(2):
        c0 = s2(2 * h, 0) + s2(2 * h + 1, 1) + s2(2 * h + 2, 2)
        c1 = s2(2 * h + 1, 0) + s2(2 * h + 2, 1) + s2(2 * h + 3, 2)
        p = jnp.maximum(c0, c1)
        p = jnp.maximum(p[:, :, 0:128], p[:, :, 128:256])
        p2.append(jnp.maximum(p + b2_ref[0], 0.0).astype(bf16))

    x3 = jnp.concatenate([z1, p2[0], z3, z1, p2[1], z3], axis=1)
    zc = jnp.zeros((tb, 16, 32), bf16)
    x3 = jnp.concatenate([zc, x3, zc], axis=2)
    z = jnp.dot(x3.reshape(tb * 16, 192), w3_ref[...],
                preferred_element_type=_F32).reshape(tb, 16, 960)
    c0 = z[:, 0:5, 0:320] + z[:, 8:13, 320:640] + z[:, 1:6, 640:960]
    c1 = z[:, 8:13, 0:320] + z[:, 1:6, 320:640] + z[:, 9:14, 640:960]
    y = jnp.maximum(c0, c1)
    y = jnp.maximum(y[:, :, 0:160], y[:, :, 160:320])
    y = jnp.maximum(y + b3_ref[0], 0.0).astype(bf16)

    xf = jnp.concatenate([y[:, i, :] for i in range(5)], axis=1)
    h = jnp.dot(xf, wf1_ref[...], preferred_element_type=_F32)
    h = jnp.maximum(h + bf1_ref[0], 0.0).astype(bf16)
    h = jnp.dot(h, wf2_ref[...], preferred_element_type=_F32)
    h = jnp.maximum(h + bf2_ref[0], 0.0).astype(bf16)
    lg = jnp.dot(h, wf3_ref[...], preferred_element_type=_F32) + bf3_ref[0]
    m = jnp.max(lg, axis=1, keepdims=True)
    lse = m + jnp.log(jnp.sum(jnp.exp(lg - m), axis=1, keepdims=True))
    o_ref[...] = lg - lse


def kernel(x_nchw, w_c1, b_c1, w_c2, b_c2, w_c3, b_c3,
           w_fc1, b_fc1, w_fc2, b_fc2, w_fc3, b_fc3):
    B = x_nchw.shape[0]
    bf16 = jnp.bfloat16

    x1 = jnp.pad(x_nchw.astype(bf16), ((0, 0), (0, 0), (2, 2), (2, 2)))
    x1 = x1.transpose(0, 2, 1, 3).reshape(B, 32, 96)
    rowperm = np.array([8 * k + g for g in range(8) for k in range(4)],
                       np.int32)
    x1 = x1[:, rowperm, :]

    w1 = _conv_toeplitz(w_c1, 3, 32, 15, True, 128).astype(bf16)
    w2 = _conv_toeplitz(w_c2, 8, 19, 8, False, 128).astype(bf16)
    w3 = _conv_toeplitz(w_c3, 16, 12, 5, False, 160).astype(bf16)

    b1 = jnp.pad(jnp.tile(b_c1.reshape(-1), 15), (0, 8)).reshape(1, 128)
    b2 = jnp.tile(b_c2.reshape(-1), 8).reshape(1, 128)
    b3 = jnp.tile(b_c3.reshape(-1), 5).reshape(1, 160)

    perm = np.array([co * 25 + py * 5 + px
                     for py in range(5) for px in range(5)
                     for co in range(32)], np.int32)
    wf1 = jnp.pad(w_fc1[perm], ((0, 0), (0, 24))).astype(bf16)
    bf1 = jnp.pad(b_fc1, ((0, 0), (0, 24)))
    wf2 = jnp.pad(w_fc2, ((0, 24), (0, 0))).astype(bf16)

    TB = 512
    G = B // TB
    nout = w_fc3.shape[1]

    out = pl.pallas_call(
        _fused_net_kernel,
        out_shape=jax.ShapeDtypeStruct((B, nout), _F32),
        grid=(G,),
        in_specs=[
            pl.BlockSpec((TB, 32, 96), lambda i: (i, 0, 0)),
            pl.BlockSpec((96, 768), lambda i: (0, 0)),
            pl.BlockSpec((1, 128), lambda i: (0, 0)),
            pl.BlockSpec((152, 768), lambda i: (0, 0)),
            pl.BlockSpec((1, 128), lambda i: (0, 0)),
            pl.BlockSpec((192, 960), lambda i: (0, 0)),
            pl.BlockSpec((1, 160), lambda i: (0, 0)),
            pl.BlockSpec((800, 1024), lambda i: (0, 0)),
            pl.BlockSpec((1, 1024), lambda i: (0, 0)),
            pl.BlockSpec((1024, 64), lambda i: (0, 0)),
            pl.BlockSpec((1, 64), lambda i: (0, 0)),
            pl.BlockSpec((64, nout), lambda i: (0, 0)),
            pl.BlockSpec((1, nout), lambda i: (0, 0)),
        ],
        out_specs=pl.BlockSpec((TB, nout), lambda i: (i, 0)),
        compiler_params=pltpu.CompilerParams(
            dimension_semantics=("parallel",)),
    )(x1, w1, b1, w2, b2, w3, b3, wf1, bf1, wf2, b_fc2,
      w_fc3.astype(bf16), b_fc3)
    return out


# speedup vs baseline: 1.0274x; 1.0274x over previous
import numpy as np
import jax
import jax.numpy as jnp
from jax.experimental import pallas as pl
from jax.experimental.pallas import tpu as pltpu

_F32 = jnp.float32


def _shift_matrix(win, pout):
    e = np.zeros((3, win, 2, pout), np.float32)
    for dx in range(3):
        for q in range(2):
            for p in range(pout):
                e[dx, 2 * p + q + dx, q, p] = 1.0
    return e


def _conv_toeplitz(w_mat, cin, win, pout, cin_major, pad_lanes):
    cout = w_mat.shape[1]
    taps = w_mat.reshape(3, 3, cin, cout)
    e = jnp.asarray(_shift_matrix(win, pout))
    blocks = []
    for dy in range(3):
        t = taps[dy]
        if cin_major:
            wd = jnp.einsum("dxqp,dco->cxqpo", e, t)
            wd = wd.reshape(cin * win, 2, pout * cout)
        else:
            wd = jnp.einsum("dxqp,dco->xcqpo", e, t)
            wd = wd.reshape(win * cin, 2, pout * cout)
        if pad_lanes > pout * cout:
            wd = jnp.pad(wd, ((0, 0), (0, 0), (0, pad_lanes - pout * cout)))
        blocks.append(wd.reshape(wd.shape[0], -1))
    return jnp.concatenate(blocks, axis=1)


def _fused_net_kernel(x_ref, w1_ref, b1_ref, w2_ref, b2_ref, w3_ref, b3_ref,
                      wf1_ref, bf1_ref, wf2_ref, bf2_ref, wf3_ref, bf3_ref,
                      o_ref):
    tb = x_ref.shape[0]
    bf16 = jnp.bfloat16

    x = x_ref[...].reshape(tb * 32, 96)
    z = jnp.dot(x, w1_ref[...], preferred_element_type=_F32)
    z = z.reshape(tb, 32, 768)

    def s1(t, b, n):
        g, ko = t % 8, t // 8
        return z[:, g * 4 + ko: g * 4 + ko + n, 256 * b:256 * (b + 1)]

    p1 = []
    for h in range(4):
        n = 3 if h == 3 else 4
        c0 = s1(2 * h, 0, n) + s1(2 * h + 1, 1, n) + s1(2 * h + 2, 2, n)
        c1 = s1(2 * h + 1, 0, n) + s1(2 * h + 2, 1, n) + s1(2 * h + 3, 2, n)
        p = jnp.maximum(c0, c1)
        p = jnp.maximum(p[:, :, 0:128], p[:, :, 128:256])
        p1.append(jnp.maximum(p + b1_ref[0], 0.0).astype(bf16))

    z1 = jnp.zeros((tb, 1, 128), bf16)
    z3 = jnp.zeros((tb, 3, 128), bf16)
    z4 = jnp.zeros((tb, 4, 128), bf16)
    x2 = jnp.concatenate(
        [z1, p1[2], z3,
         z1, p1[3], z4,
         p1[0], z4,
         p1[1], z4], axis=1)
    zc = jnp.zeros((tb, 32, 16), bf16)
    x2 = jnp.concatenate([zc, x2, zc[:, :, 0:8]], axis=2)
    z = jnp.dot(x2.reshape(tb * 32, 152), w2_ref[...],
                preferred_element_type=_F32).reshape(tb, 32, 768)

    def s2(t, b):
        g, ko = t % 4, t // 4
        return z[:, g * 8 + ko: g * 8 + ko + 4, 256 * b:256 * (b + 1)]

    p2 = []
    for h in range(2):
        c0 = s2(2 * h, 0) + s2(2 * h + 1, 1) + s2(2 * h + 2, 2)
        c1 = s2(2 * h + 1, 0) + s2(2 * h + 2, 1) + s2(2 * h + 3, 2)
        p = jnp.maximum(c0, c1)
        p = jnp.maximum(p[:, :, 0:128], p[:, :, 128:256])
        p2.append(jnp.maximum(p + b2_ref[0], 0.0).astype(bf16))

    x3 = jnp.concatenate([z1, p2[0], z3, z1, p2[1], z3], axis=1)
    zc = jnp.zeros((tb, 16, 32), bf16)
    x3 = jnp.concatenate([zc, x3, zc], axis=2)
    z = jnp.dot(x3.reshape(tb * 16, 192), w3_ref[...],
                preferred_element_type=_F32).reshape(tb, 16, 960)
    c0 = z[:, 0:5, 0:320] + z[:, 8:13, 320:640] + z[:, 1:6, 640:960]
    c1 = z[:, 8:13, 0:320] + z[:, 1:6, 320:640] + z[:, 9:14, 640:960]
    y = jnp.maximum(c0, c1)
    y = jnp.maximum(y[:, :, 0:160], y[:, :, 160:320])
    y = jnp.maximum(y + b3_ref[0], 0.0).astype(bf16)

    xf = jnp.concatenate([y[:, i, :] for i in range(5)], axis=1)
    h = jnp.dot(xf, wf1_ref[...], preferred_element_type=_F32)
    h = jnp.maximum(h + bf1_ref[0], 0.0).astype(bf16)
    h = jnp.dot(h, wf2_ref[...], preferred_element_type=_F32)
    h = jnp.maximum(h + bf2_ref[0], 0.0).astype(bf16)
    lg = jnp.dot(h, wf3_ref[...], preferred_element_type=_F32) + bf3_ref[0]
    m = jnp.max(lg, axis=1, keepdims=True)
    lse = m + jnp.log(jnp.sum(jnp.exp(lg - m), axis=1, keepdims=True))
    o_ref[...] = lg - lse


def kernel(x_nchw, w_c1, b_c1, w_c2, b_c2, w_c3, b_c3,
           w_fc1, b_fc1, w_fc2, b_fc2, w_fc3, b_fc3):
    B = x_nchw.shape[0]
    bf16 = jnp.bfloat16

    x1 = jnp.pad(x_nchw.astype(bf16), ((0, 0), (0, 0), (2, 2), (2, 2)))
    x1 = x1.transpose(0, 2, 1, 3).reshape(B, 32, 96)
    rowperm = np.array([8 * k + g for g in range(8) for k in range(4)],
                       np.int32)
    x1 = x1[:, rowperm, :]

    w1 = _conv_toeplitz(w_c1, 3, 32, 15, True, 128).astype(bf16)
    w2 = _conv_toeplitz(w_c2, 8, 19, 8, False, 128).astype(bf16)
    w3 = _conv_toeplitz(w_c3, 16, 12, 5, False, 160).astype(bf16)

    b1 = jnp.pad(jnp.tile(b_c1.reshape(-1), 15), (0, 8)).reshape(1, 128)
    b2 = jnp.tile(b_c2.reshape(-1), 8).reshape(1, 128)
    b3 = jnp.tile(b_c3.reshape(-1), 5).reshape(1, 160)

    perm = np.array([co * 25 + py * 5 + px
                     for py in range(5) for px in range(5)
                     for co in range(32)], np.int32)
    wf1 = jnp.pad(w_fc1[perm], ((0, 0), (0, 24))).astype(bf16)
    bf1 = jnp.pad(b_fc1, ((0, 0), (0, 24)))
    wf2 = jnp.pad(w_fc2, ((0, 24), (0, 0))).astype(bf16)

    TB = 256
    G = B // TB
    nout = w_fc3.shape[1]

    out = pl.pallas_call(
        _fused_net_kernel,
        out_shape=jax.ShapeDtypeStruct((B, nout), _F32),
        grid=(G,),
        in_specs=[
            pl.BlockSpec((TB, 32, 96), lambda i: (i, 0, 0)),
            pl.BlockSpec((96, 768), lambda i: (0, 0)),
            pl.BlockSpec((1, 128), lambda i: (0, 0)),
            pl.BlockSpec((152, 768), lambda i: (0, 0)),
            pl.BlockSpec((1, 128), lambda i: (0, 0)),
            pl.BlockSpec((192, 960), lambda i: (0, 0)),
            pl.BlockSpec((1, 160), lambda i: (0, 0)),
            pl.BlockSpec((800, 1024), lambda i: (0, 0)),
            pl.BlockSpec((1, 1024), lambda i: (0, 0)),
            pl.BlockSpec((1024, 64), lambda i: (0, 0)),
            pl.BlockSpec((1, 64), lambda i: (0, 0)),
            pl.BlockSpec((64, nout), lambda i: (0, 0)),
            pl.BlockSpec((1, nout), lambda i: (0, 0)),
        ],
        out_specs=pl.BlockSpec((TB, nout), lambda i: (i, 0)),
        compiler_params=pltpu.CompilerParams(
            dimension_semantics=("parallel",)),
    )(x1, w1, b1, w2, b2, w3, b3, wf1, bf1, wf2, b_fc2,
      w_fc3.astype(bf16), b_fc3)
    return out
